# baseline (speedup 1.0000x reference)
import jax
import jax.numpy as jnp
from jax.experimental import pallas as pl


def _mean_agg(x, src, dst, n):
    s = jax.ops.segment_sum(x[src], dst, num_segments=n)
    cnt = jax.ops.segment_sum(jnp.ones((src.shape[0],), x.dtype), dst, num_segments=n)
    return s / jnp.maximum(cnt, 1.0)[:, None]


def _gcn_agg(x, src, dst, n):
    ones = jnp.ones((src.shape[0],), x.dtype)
    deg = jax.ops.segment_sum(ones, dst, num_segments=n) + 1.0
    norm = jax.lax.rsqrt(deg[src] * deg[dst])
    s = jax.ops.segment_sum(x[src] * norm[:, None], dst, num_segments=n)
    return s + x / deg[:, None]


def _copy_kernel(x_ref, o_ref):
    o_ref[...] = x_ref[...]


def _pallas_copy(x):
    return pl.pallas_call(
        _copy_kernel,
        out_shape=jax.ShapeDtypeStruct(x.shape, x.dtype),
    )(x)


def kernel(node_features, euc_edge_index, sph_edge_index, hgc_edge_index, target_node_idx,
           W1_self, W1_nei, b1, W2_self, W2_nei, b2,
           Wg1, bg1, Wg2, bg2, Wh1, bh1, Wh2, bh2):
    n = node_features.shape[0]
    es, ed = euc_edge_index[0], euc_edge_index[1]
    h = jax.nn.relu(node_features @ W1_self + _mean_agg(node_features @ W1_nei, es, ed, n) + b1)
    euc = h @ W2_self + _mean_agg(h @ W2_nei, es, ed, n) + b2
    ss, sd = sph_edge_index[0], sph_edge_index[1]
    h = jax.nn.relu(_gcn_agg(node_features @ Wg1, ss, sd, n) + bg1)
    nrm = jnp.maximum(jnp.linalg.norm(_gcn_agg(h @ Wg2, ss, sd, n) + bg2, axis=-1, keepdims=True), 1e-6)
    sph = (_gcn_agg(h @ Wg2, ss, sd, n) + bg2) / nrm
    hs, hd = hgc_edge_index[0], hgc_edge_index[1]
    h = jax.nn.relu(_gcn_agg(node_features @ Wh1, hs, hd, n) + bh1)
    u = _gcn_agg(h @ Wh2, hs, hd, n) + bh2
    unrm = jnp.maximum(jnp.linalg.norm(u, axis=-1, keepdims=True), 1e-6)
    hgc = jnp.tanh(unrm) * u / unrm
    return _pallas_copy(jnp.stack([euc, sph, hgc]))


# speedup vs baseline: 1.3928x; 1.3928x over previous
import functools

import jax
import jax.numpy as jnp
from jax.experimental import pallas as pl
from jax.experimental.pallas import tpu as pltpu

N = 10000
IN_DIM = 1433
HID = 512
OUT = 256
BM = 1000


def _segsum(table, src, dst, n):
    return jax.ops.segment_sum(table[src], dst, num_segments=n)


def _hist(dst, n):
    return jax.ops.segment_sum(jnp.ones(dst.shape, jnp.float32), dst, num_segments=n)


def _m1_body(x_ref, w_ref, rss_ref, rsh_ref, a1_ref, b1_ref, g1_ref, h1_ref):
    acc = jnp.dot(x_ref[...], w_ref[...], preferred_element_type=jnp.float32)
    a1_ref[...] = acc[:, 0:HID]
    b1_ref[...] = acc[:, HID:2 * HID]
    g1_ref[...] = acc[:, 2 * HID:3 * HID] * rss_ref[...]
    h1_ref[...] = acc[:, 3 * HID:4 * HID] * rsh_ref[...]


def _m1(x, wcat, rs_s, rs_h):
    grid = (N // BM,)
    o = jax.ShapeDtypeStruct((N, HID), jnp.float32)
    return pl.pallas_call(
        _m1_body,
        grid=grid,
        in_specs=[
            pl.BlockSpec((BM, IN_DIM), lambda i: (i, 0)),
            pl.BlockSpec((IN_DIM, 4 * HID), lambda i: (0, 0)),
            pl.BlockSpec((BM, 1), lambda i: (i, 0)),
            pl.BlockSpec((BM, 1), lambda i: (i, 0)),
        ],
        out_specs=[pl.BlockSpec((BM, HID), lambda i: (i, 0))] * 4,
        out_shape=[o, o, o, o],
    )(x, wcat, rs_s, rs_h)


def _l2e_body(a1_ref, s1_ref, ic_ref, b1_ref, w_ref, a2_ref, b2_ref):
    h = jax.nn.relu(a1_ref[...] + s1_ref[...] * ic_ref[...] + b1_ref[...])
    acc = jnp.dot(h, w_ref[...], preferred_element_type=jnp.float32)
    a2_ref[...] = acc[:, 0:OUT]
    b2_ref[...] = acc[:, OUT:2 * OUT]


def _l2_euc(a1, s1, invcnt, b1, w2cat):
    o = jax.ShapeDtypeStruct((N, OUT), jnp.float32)
    return pl.pallas_call(
        _l2e_body,
        grid=(N // BM,),
        in_specs=[
            pl.BlockSpec((BM, HID), lambda i: (i, 0)),
            pl.BlockSpec((BM, HID), lambda i: (i, 0)),
            pl.BlockSpec((BM, 1), lambda i: (i, 0)),
            pl.BlockSpec((1, HID), lambda i: (0, 0)),
            pl.BlockSpec((HID, 2 * OUT), lambda i: (0, 0)),
        ],
        out_specs=[pl.BlockSpec((BM, OUT), lambda i: (i, 0))] * 2,
        out_shape=[o, o],
    )(a1, s1, invcnt, b1.reshape(1, HID), w2cat)


def _l2g_body(s1_ref, g1_ref, rs_ref, bg_ref, w_ref, g2_ref):
    rs = rs_ref[...]
    h = jax.nn.relu((s1_ref[...] + g1_ref[...]) * rs + bg_ref[...])
    g2_ref[...] = jnp.dot(h, w_ref[...], preferred_element_type=jnp.float32) * rs


def _l2_gcn(s1, g1p, rs, bg1, wg2):
    return pl.pallas_call(
        _l2g_body,
        grid=(N // BM,),
        in_specs=[
            pl.BlockSpec((BM, HID), lambda i: (i, 0)),
            pl.BlockSpec((BM, HID), lambda i: (i, 0)),
            pl.BlockSpec((BM, 1), lambda i: (i, 0)),
            pl.BlockSpec((1, HID), lambda i: (0, 0)),
            pl.BlockSpec((HID, OUT), lambda i: (0, 0)),
        ],
        out_specs=pl.BlockSpec((BM, OUT), lambda i: (i, 0)),
        out_shape=jax.ShapeDtypeStruct((N, OUT), jnp.float32),
    )(s1, g1p, rs, bg1.reshape(1, HID), wg2)


def _fin_body(a2_ref, s2e_ref, ic_ref, b2_ref,
              s2s_ref, g2_ref, rss_ref, bg2_ref,
              s2h_ref, h2_ref, rsh_ref, bh2_ref, out_ref):
    euc = a2_ref[...] + s2e_ref[...] * ic_ref[...] + b2_ref[...]
    sph_pre = (s2s_ref[...] + g2_ref[...]) * rss_ref[...] + bg2_ref[...]
    n1 = jnp.maximum(jnp.sqrt(jnp.sum(sph_pre * sph_pre, axis=-1, keepdims=True)), 1e-6)
    sph = sph_pre / n1
    u = (s2h_ref[...] + h2_ref[...]) * rsh_ref[...] + bh2_ref[...]
    n2 = jnp.maximum(jnp.sqrt(jnp.sum(u * u, axis=-1, keepdims=True)), 1e-6)
    hgc = jnp.tanh(n2) * u / n2
    out_ref[0, :, :] = euc
    out_ref[1, :, :] = sph
    out_ref[2, :, :] = hgc


def _final(a2, s2e, invcnt, b2, s2s, g2p, rs_s, bg2, s2h, h2p, rs_h, bh2):
    bmat = pl.BlockSpec((BM, OUT), lambda i: (i, 0))
    brow = pl.BlockSpec((BM, 1), lambda i: (i, 0))
    bb = pl.BlockSpec((1, OUT), lambda i: (0, 0))
    return pl.pallas_call(
        _fin_body,
        grid=(N // BM,),
        in_specs=[bmat, bmat, brow, bb, bmat, bmat, brow, bb, bmat, bmat, brow, bb],
        out_specs=pl.BlockSpec((3, BM, OUT), lambda i: (0, i, 0)),
        out_shape=jax.ShapeDtypeStruct((3, N, OUT), jnp.float32),
    )(a2, s2e, invcnt, b2.reshape(1, OUT), s2s, g2p, rs_s, bg2.reshape(1, OUT),
      s2h, h2p, rs_h, bh2.reshape(1, OUT))


def kernel(node_features, euc_edge_index, sph_edge_index, hgc_edge_index, target_node_idx,
           W1_self, W1_nei, b1, W2_self, W2_nei, b2,
           Wg1, bg1, Wg2, bg2, Wh1, bh1, Wh2, bh2):
    es, ed = euc_edge_index[0], euc_edge_index[1]
    ss, sd = sph_edge_index[0], sph_edge_index[1]
    hs, hd = hgc_edge_index[0], hgc_edge_index[1]

    cnt_e = _hist(ed, N)
    invcnt = (1.0 / jnp.maximum(cnt_e, 1.0)).reshape(N, 1)
    deg_s = _hist(sd, N) + 1.0
    deg_h = _hist(hd, N) + 1.0
    rs_s = jax.lax.rsqrt(deg_s).reshape(N, 1)
    rs_h = jax.lax.rsqrt(deg_h).reshape(N, 1)

    wcat = jnp.concatenate([W1_self, W1_nei, Wg1, Wh1], axis=1)
    a1, b1t, g1p, h1p = _m1(node_features, wcat, rs_s, rs_h)

    s1e = _segsum(b1t, es, ed, N)
    s1s = _segsum(g1p, ss, sd, N)
    s1h = _segsum(h1p, hs, hd, N)

    w2cat = jnp.concatenate([W2_self, W2_nei], axis=1)
    a2, b2t = _l2_euc(a1, s1e, invcnt, b1, w2cat)
    g2p = _l2_gcn(s1s, g1p, rs_s, bg1, Wg2)
    h2p = _l2_gcn(s1h, h1p, rs_h, bh1, Wh2)

    s2e = _segsum(b2t, es, ed, N)
    s2s = _segsum(g2p, ss, sd, N)
    s2h = _segsum(h2p, hs, hd, N)

    return _final(a2, s2e, invcnt, b2, s2s, g2p, rs_s, bg2, s2h, h2p, rs_h, bh2)
